# baseline (speedup 1.0000x reference)
import functools

import jax
import jax.numpy as jnp
from jax import lax
from jax.experimental import pallas as pl
from jax.experimental.pallas import tpu as pltpu
from jax.experimental.pallas import tpu_sc as plsc

_B, _S, _D = 4, 2048, 4096
_L = 16
_CHUNKS = _S // _L

_mesh = plsc.VectorSubcoreMesh(core_axis_name="c", subcore_axis_name="s",
                               num_cores=1, num_subcores=4)


@functools.partial(
    pl.kernel,
    mesh=_mesh,
    out_type=jax.ShapeDtypeStruct((_B, _L), jnp.float32),
    scratch_types=[
        pltpu.VMEM((_S,), jnp.int32),
        pltpu.VMEM((_L,), jnp.float32),
        pltpu.VMEM((_L,), jnp.float32),
    ],
)
def _sc_argmax_pick(ids_hbm, lhs_hbm, out_hbm, ids_v, row_v, out_v):
    wid = lax.axis_index("s") + lax.axis_index("c")

    @pl.when(wid < _B)
    def _():
        b = wid
        pltpu.sync_copy(ids_hbm.at[b], ids_v)
        lanes = lax.iota(jnp.int32, _L)
        rev_lanes = (_S - 1) - lanes

        _ACC = 8
        accs = [jnp.full((_L,), jnp.iinfo(jnp.int32).min, jnp.int32)
                for _ in range(_ACC)]
        for c in range(_CHUNKS):
            v = ids_v[pl.ds(c * _L, _L)]
            comb = v * _S + (rev_lanes - c * _L)
            accs[c % _ACC] = jnp.maximum(accs[c % _ACC], comb)
        while len(accs) > 1:
            accs = [jnp.maximum(accs[i], accs[i + len(accs) // 2])
                    for i in range(len(accs) // 2)]
        best = accs[0]

        for sh in (8, 4, 2, 1):
            perm = (lanes + sh) & (_L - 1)
            best = jnp.maximum(
                best, best.at[perm].get(mode="promise_in_bounds"))

        bestk = best[0]
        idx = (_S - 1) - lax.rem(bestk, _S)

        base = (idx // _L) * _L
        pltpu.sync_copy(lhs_hbm.at[b, idx, pl.ds(base, _L)], row_v)
        sel = row_v[...].at[jnp.broadcast_to(idx - base, (_L,))].get(
            mode="promise_in_bounds")
        out_v[...] = sel * sel
        pltpu.sync_copy(out_v, out_hbm.at[b])


def kernel(last_hidden_state, input_ids):
    ids = input_ids.astype(jnp.int32)
    out = _sc_argmax_pick(ids, last_hidden_state)
    return out[:, 0]


# speedup vs baseline: 1.0296x; 1.0296x over previous
import functools

import jax
import jax.numpy as jnp
from jax import lax
from jax.experimental import pallas as pl
from jax.experimental.pallas import tpu as pltpu
from jax.experimental.pallas import tpu_sc as plsc
from jax._src.pallas import mpmd

_B, _S, _D = 4, 2048, 4096
_L = 16
_CHUNKS = _S // _L

_smesh = plsc.ScalarSubcoreMesh(axis_name="c", num_cores=1)
_vmesh = plsc.VectorSubcoreMesh(core_axis_name="c", subcore_axis_name="s",
                                num_cores=1, num_subcores=4)
_vv = pltpu.VMEM @ _vmesh


def _scs_fn(ids_hbm, lhs_hbm, out_hbm, ids_sh, ids_v, row_v, out_v, sem):
    pltpu.sync_copy(ids_hbm, ids_sh)
    for t in range(_B):
        pltpu.semaphore_signal(sem, 1, device_id={"s": t})


def _tec_fn(ids_hbm, lhs_hbm, out_hbm, ids_sh, ids_v, row_v, out_v, sem):
    wid = lax.axis_index("s") + lax.axis_index("c")

    @pl.when(wid < _B)
    def _():
        b = wid
        pltpu.semaphore_wait(sem, 1)
        pltpu.sync_copy(ids_sh.at[b], ids_v)
        lanes = lax.iota(jnp.int32, _L)
        rev_lanes = (_S - 1) - lanes

        _ACC = 8
        accs = [jnp.full((_L,), jnp.iinfo(jnp.int32).min, jnp.int32)
                for _ in range(_ACC)]
        for c in range(_CHUNKS):
            v = ids_v[pl.ds(c * _L, _L)]
            comb = v * _S + (rev_lanes - c * _L)
            accs[c % _ACC] = jnp.maximum(accs[c % _ACC], comb)
        while len(accs) > 1:
            accs = [jnp.maximum(accs[i], accs[i + len(accs) // 2])
                    for i in range(len(accs) // 2)]
        best = accs[0]

        for sh in (8, 4, 2, 1):
            perm = (lanes + sh) & (_L - 1)
            best = jnp.maximum(
                best, best.at[perm].get(mode="promise_in_bounds"))

        bestk = best[0]
        idx = (_S - 1) - lax.rem(bestk, _S)

        base = (idx // _L) * _L
        pltpu.sync_copy(lhs_hbm.at[b, idx, pl.ds(base, _L)], row_v)
        sel = row_v[...].at[jnp.broadcast_to(idx - base, (_L,))].get(
            mode="promise_in_bounds")
        out_v[...] = sel * sel
        pltpu.sync_copy(out_v, out_hbm.at[b])


_call = mpmd.mpmd_map(
    [(_smesh, _scs_fn), (_vmesh, _tec_fn)],
    out_types=jax.ShapeDtypeStruct((_B, _L), jnp.float32),
    scratch_types=[
        pltpu.VMEM_SHARED((_B, _S), jnp.int32),
        _vv((_S,), jnp.int32),
        _vv((_L,), jnp.float32),
        _vv((_L,), jnp.float32),
        pltpu.SemaphoreType.REGULAR @ _vmesh,
    ],
)


def kernel(last_hidden_state, input_ids):
    ids = input_ids.astype(jnp.int32)
    out = _call(ids, last_hidden_state)
    return out[:, 0]


# speedup vs baseline: 1.0313x; 1.0016x over previous
import functools

import jax
import jax.numpy as jnp
from jax import lax
from jax.experimental import pallas as pl
from jax.experimental.pallas import tpu as pltpu
from jax.experimental.pallas import tpu_sc as plsc
from jax._src.pallas import mpmd

_B, _S, _D = 4, 2048, 4096
_L = 16
_CHUNKS = _S // _L

_smesh = plsc.ScalarSubcoreMesh(axis_name="c", num_cores=1)
_vmesh = plsc.VectorSubcoreMesh(core_axis_name="c", subcore_axis_name="s",
                                num_cores=1, num_subcores=_B)
_vv = pltpu.VMEM @ _vmesh


def _scs_stage(ids_hbm, lhs_hbm, out_hbm, ids_sh, ids_v, row_v, out_v, sem):
    del lhs_hbm, out_hbm, ids_v, row_v, out_v
    pltpu.sync_copy(ids_hbm, ids_sh)
    for t in range(_B):
        pltpu.semaphore_signal(sem, 1, device_id={"s": t})


def _tec_argmax_pick(ids_hbm, lhs_hbm, out_hbm, ids_sh, ids_v, row_v, out_v,
                     sem):
    del ids_hbm
    wid = lax.axis_index("s") + lax.axis_index("c")

    @pl.when(wid < _B)
    def _():
        b = wid
        pltpu.semaphore_wait(sem, 1)
        pltpu.sync_copy(ids_sh.at[b], ids_v)
        lanes = lax.iota(jnp.int32, _L)

        _ACC = 8
        accs = [jnp.full((_L,), jnp.iinfo(jnp.int32).min, jnp.int32)
                for _ in range(_ACC)]
        wc = (_S - 1) - lanes
        for c in range(_CHUNKS):
            v = ids_v[pl.ds(c * _L, _L)]
            accs[c % _ACC] = jnp.maximum(accs[c % _ACC], v * _S + wc)
            wc = wc - _L
        while len(accs) > 1:
            accs = [jnp.maximum(accs[i], accs[i + len(accs) // 2])
                    for i in range(len(accs) // 2)]
        best = accs[0]

        for sh in (8, 4, 2, 1):
            perm = (lanes + sh) & (_L - 1)
            best = jnp.maximum(
                best, best.at[perm].get(mode="promise_in_bounds"))

        bestk = best[0]
        idx = (_S - 1) - lax.rem(bestk, _S)

        base = (idx // _L) * _L
        pltpu.sync_copy(lhs_hbm.at[b, idx, pl.ds(base, _L)], row_v)
        sel = row_v[...].at[jnp.broadcast_to(idx - base, (_L,))].get(
            mode="promise_in_bounds")
        out_v[...] = sel * sel
        pltpu.sync_copy(out_v, out_hbm.at[b])


_sc_call = mpmd.mpmd_map(
    [(_smesh, _scs_stage), (_vmesh, _tec_argmax_pick)],
    out_types=jax.ShapeDtypeStruct((_B, _L), jnp.float32),
    scratch_types=[
        pltpu.VMEM_SHARED((_B, _S), jnp.int32),
        _vv((_S,), jnp.int32),
        _vv((_L,), jnp.float32),
        _vv((_L,), jnp.float32),
        pltpu.SemaphoreType.REGULAR @ _vmesh,
    ],
)


def kernel(last_hidden_state, input_ids):
    ids = input_ids.astype(jnp.int32)
    out = _sc_call(ids, last_hidden_state)
    return out[:, 0]
